# baseline (speedup 1.0000x reference)
import jax
import jax.numpy as jnp
from jax import lax
from jax.experimental import pallas as pl
from jax.experimental.pallas import tpu as pltpu


def _mbl_body(boxes_ref, labels_ref, priors_ref, locs_ref, scores_ref,
              out_ref, acc_ref):
    g = pl.program_id(0)

    boxes_g = boxes_ref[...]
    labels_g = labels_ref[...]
    pr = priors_ref[...]
    locs4 = locs_ref[...]
    s = scores_ref[...]
    n_img, nobj, _ = boxes_g.shape
    n_cls = s.shape[0]
    n_pri = s.shape[2]

    px0 = pr[0:1]
    py0 = pr[1:2]
    px1 = pr[2:3]
    py1 = pr[3:4]
    pcx = pr[4:5]
    pcy = pr[5:6]
    area_p = pr[6:7]
    m10w = pr[7:8]
    m10h = pr[8:9]
    lpw5 = pr[9:10]
    lph5 = pr[10:11]

    bgx = (jnp.reshape(boxes_g[:, :, 0:1], (n_img, nobj)) +
           jnp.reshape(boxes_g[:, :, 2:3], (n_img, nobj))) / 2.0
    bgy = (jnp.reshape(boxes_g[:, :, 1:2], (n_img, nobj)) +
           jnp.reshape(boxes_g[:, :, 3:4], (n_img, nobj))) / 2.0
    blw5 = 5.0 * jnp.log(jnp.reshape(boxes_g[:, :, 2:3], (n_img, nobj)) -
                         jnp.reshape(boxes_g[:, :, 0:1], (n_img, nobj)))
    blh5 = 5.0 * jnp.log(jnp.reshape(boxes_g[:, :, 3:4], (n_img, nobj)) -
                         jnp.reshape(boxes_g[:, :, 1:2], (n_img, nobj)))

    obj_iota = lax.broadcasted_iota(jnp.int32, (nobj, n_pri), 0)
    p_iota = lax.broadcasted_iota(jnp.int32, (nobj, n_pri), 1)

    ofe_rows = []
    ovl_rows = []
    for j in range(n_img):
        bj = boxes_g[j]
        bx0 = bj[:, 0:1]
        by0 = bj[:, 1:2]
        bx1 = bj[:, 2:3]
        by1 = bj[:, 3:4]
        iw = jnp.clip(jnp.minimum(bx1, px1) - jnp.maximum(bx0, px0), 0.0)
        ih = jnp.clip(jnp.minimum(by1, py1) - jnp.maximum(by0, py0), 0.0)
        inter = iw * ih
        union = (bx1 - bx0) * (by1 - by0) + area_p - inter
        iou = inter / union

        ovl = jnp.max(iou, axis=0, keepdims=True)
        ofe = jnp.min(jnp.where(iou == ovl, obj_iota, nobj), axis=0,
                      keepdims=True)
        row_max = jnp.max(iou, axis=1, keepdims=True)
        pfo = jnp.min(jnp.where(iou == row_max, p_iota, n_pri), axis=1,
                      keepdims=True)
        hit = p_iota == pfo
        best_j = jnp.max(jnp.where(hit, obj_iota, -1), axis=0,
                         keepdims=True)
        forced = best_j >= 0
        ofe_rows.append(jnp.where(forced, best_j, ofe))
        ovl_rows.append(jnp.where(forced, 1.0, ovl))

    ofe8 = jnp.concatenate(ofe_rows, axis=0)
    ovl8 = jnp.concatenate(ovl_rows, axis=0)

    lab = jnp.zeros((n_img, n_pri), jnp.int32)
    gcx = jnp.zeros((n_img, n_pri), jnp.float32)
    gcy = jnp.zeros((n_img, n_pri), jnp.float32)
    glw = jnp.zeros((n_img, n_pri), jnp.float32)
    glh = jnp.zeros((n_img, n_pri), jnp.float32)
    for j in range(nobj):
        sel = ofe8 == j
        lab = jnp.where(sel, labels_g[:, j:j + 1], lab)
        gcx = jnp.where(sel, bgx[:, j:j + 1], gcx)
        gcy = jnp.where(sel, bgy[:, j:j + 1], gcy)
        glw = jnp.where(sel, blw5[:, j:j + 1], glw)
        glh = jnp.where(sel, blh5[:, j:j + 1], glh)

    lab = jnp.where(ovl8 < 0.5, 0, lab)
    pos = lab != 0
    posf = pos.astype(jnp.float32)
    npos8 = jnp.sum(posf, axis=1, keepdims=True)
    n_pos = jnp.sum(npos8)

    t0 = (gcx - pcx) * m10w
    t1 = (gcy - pcy) * m10h
    t2 = glw - lpw5
    t3 = glh - lph5
    loc_sum = (jnp.sum(jnp.abs(locs4[0] - t0) * posf) +
               jnp.sum(jnp.abs(locs4[1] - t1) * posf) +
               jnp.sum(jnp.abs(locs4[2] - t2) * posf) +
               jnp.sum(jnp.abs(locs4[3] - t3) * posf))

    m = s[0]
    for c in range(1, n_cls):
        m = jnp.maximum(m, s[c])
    se = jnp.exp(s[0] - m)
    for c in range(1, n_cls):
        se = se + jnp.exp(s[c] - m)
    lse = jnp.log(se) + m
    picked = jnp.zeros((n_img, n_pri), jnp.float32)
    for c in range(n_cls):
        picked = jnp.where(lab == c, s[c], picked)
    conf_all = lse - picked
    conf_pos = jnp.sum(conf_all * posf)
    conf_neg = jnp.where(pos, 0.0, conf_all)

    kf8 = 3.0 * npos8
    v = lax.bitcast_convert_type(conf_neg, jnp.int32)
    t8 = jnp.zeros((n_img, 1), jnp.int32)
    for b in range(30, -1, -1):
        cand = t8 | jnp.int32(1 << b)
        cnt = jnp.sum(jnp.where(v >= cand, 1.0, 0.0), axis=1,
                      keepdims=True)
        t8 = jnp.where(cnt >= kf8, cand, t8)
    tf8 = lax.bitcast_convert_type(t8, jnp.float32)
    gt = v > t8
    cnt_gt = jnp.sum(jnp.where(gt, 1.0, 0.0), axis=1, keepdims=True)
    sum_gt = jnp.sum(jnp.where(gt, conf_neg, 0.0), axis=1, keepdims=True)
    hard8 = jnp.where(kf8 > 0.0, sum_gt + (kf8 - cnt_gt) * tf8, 0.0)
    hard = jnp.sum(hard8)

    @pl.when(g == 0)
    def _init():
        acc_ref[0] = loc_sum
        acc_ref[1] = conf_pos
        acc_ref[2] = hard
        acc_ref[3] = n_pos

    @pl.when(g > 0)
    def _acc():
        acc_ref[0] += loc_sum
        acc_ref[1] += conf_pos
        acc_ref[2] += hard
        acc_ref[3] += n_pos

    n_tot = acc_ref[3]
    loss = (acc_ref[2] + acc_ref[1]) / n_tot + acc_ref[0] / (n_tot * 4.0)
    out_ref[...] = jnp.full((8, 128), loss, jnp.float32)


@jax.jit
def kernel(predicted_locs, predicted_scores, boxes, labels, priors_cxcy):
    B, P, C = predicted_scores.shape
    nobj = boxes.shape[1]
    ipg = 8
    while B % ipg:
        ipg //= 2
    locs_t = jnp.transpose(predicted_locs.astype(jnp.float32), (2, 0, 1))
    scores_t = jnp.transpose(predicted_scores.astype(jnp.float32), (2, 0, 1))
    pcx, pcy, pw, ph = [priors_cxcy.astype(jnp.float32)[:, c] for c in range(4)]
    px0, py0 = pcx - pw / 2.0, pcy - ph / 2.0
    px1, py1 = pcx + pw / 2.0, pcy + ph / 2.0
    priors_t = jnp.stack([
        px0, py0, px1, py1, pcx, pcy,
        (px1 - px0) * (py1 - py0), 10.0 / pw, 10.0 / ph,
        5.0 * jnp.log(pw), 5.0 * jnp.log(ph),
    ], axis=0)
    labels_i = labels.astype(jnp.int32)
    boxes_f = boxes.astype(jnp.float32)

    out = pl.pallas_call(
        _mbl_body,
        grid=(B // ipg,),
        in_specs=[
            pl.BlockSpec((ipg, nobj, 4), lambda i: (i, 0, 0)),
            pl.BlockSpec((ipg, nobj), lambda i: (i, 0)),
            pl.BlockSpec((11, P), lambda i: (0, 0)),
            pl.BlockSpec((4, ipg, P), lambda i: (0, i, 0)),
            pl.BlockSpec((C, ipg, P), lambda i: (0, i, 0)),
        ],
        out_specs=pl.BlockSpec((8, 128), lambda i: (0, 0)),
        out_shape=jax.ShapeDtypeStruct((8, 128), jnp.float32),
        scratch_shapes=[pltpu.SMEM((4,), jnp.float32)],
    )(boxes_f, labels_i, priors_t, locs_t, scores_t)
    return out[0, 0]


# speedup vs baseline: 1.0891x; 1.0891x over previous
import jax
import jax.numpy as jnp
from jax import lax
from jax.experimental import pallas as pl
from jax.experimental.pallas import tpu as pltpu


def _mbl_body(boxes_ref, labels_ref, priors_ref, locs_ref, scores_ref,
              out_ref, acc_ref):
    g = pl.program_id(0)

    boxes_g = boxes_ref[...]
    labels_g = labels_ref[...]
    pr = priors_ref[...]
    locs4 = locs_ref[...]
    s = scores_ref[...]
    n_img, nobj, _ = boxes_g.shape
    n_cls = s.shape[0]
    n_pri = s.shape[2]

    pcx = pr[0:1]
    pcy = pr[1:2]
    pw = pr[2:3]
    ph = pr[3:4]
    px0 = pcx - pw / 2.0
    py0 = pcy - ph / 2.0
    px1 = pcx + pw / 2.0
    py1 = pcy + ph / 2.0
    area_p = (px1 - px0) * (py1 - py0)

    obj_iota = lax.broadcasted_iota(jnp.int32, (nobj, n_pri), 0)
    p_iota = lax.broadcasted_iota(jnp.int32, (nobj, n_pri), 1)

    ofe_rows = []
    ovl_rows = []
    for j in range(n_img):
        bj = boxes_g[j]
        bx0 = bj[:, 0:1]
        by0 = bj[:, 1:2]
        bx1 = bj[:, 2:3]
        by1 = bj[:, 3:4]
        iw = jnp.clip(jnp.minimum(bx1, px1) - jnp.maximum(bx0, px0), 0.0)
        ih = jnp.clip(jnp.minimum(by1, py1) - jnp.maximum(by0, py0), 0.0)
        inter = iw * ih
        union = (bx1 - bx0) * (by1 - by0) + area_p - inter
        iou = inter / union

        ovl = jnp.max(iou, axis=0, keepdims=True)
        ofe = jnp.min(jnp.where(iou == ovl, obj_iota, nobj), axis=0,
                      keepdims=True)
        row_max = jnp.max(iou, axis=1, keepdims=True)
        pfo = jnp.min(jnp.where(iou == row_max, p_iota, n_pri), axis=1,
                      keepdims=True)
        hit = p_iota == pfo
        best_j = jnp.max(jnp.where(hit, obj_iota, -1), axis=0,
                         keepdims=True)
        forced = best_j >= 0
        ofe_rows.append(jnp.where(forced, best_j, ofe))
        ovl_rows.append(jnp.where(forced, 1.0, ovl))

    ofe8 = jnp.concatenate(ofe_rows, axis=0)
    ovl8 = jnp.concatenate(ovl_rows, axis=0)

    lab = jnp.zeros((n_img, n_pri), jnp.int32)
    gx0 = jnp.zeros((n_img, n_pri), jnp.float32)
    gy0 = jnp.zeros((n_img, n_pri), jnp.float32)
    gx1 = jnp.zeros((n_img, n_pri), jnp.float32)
    gy1 = jnp.zeros((n_img, n_pri), jnp.float32)
    for j in range(nobj):
        sel = ofe8 == j
        lab = jnp.where(sel, labels_g[:, j:j + 1], lab)
        gx0 = jnp.where(sel, jnp.reshape(boxes_g[:, j:j + 1, 0:1],
                                         (n_img, 1)), gx0)
        gy0 = jnp.where(sel, jnp.reshape(boxes_g[:, j:j + 1, 1:2],
                                         (n_img, 1)), gy0)
        gx1 = jnp.where(sel, jnp.reshape(boxes_g[:, j:j + 1, 2:3],
                                         (n_img, 1)), gx1)
        gy1 = jnp.where(sel, jnp.reshape(boxes_g[:, j:j + 1, 3:4],
                                         (n_img, 1)), gy1)

    lab = jnp.where(ovl8 < 0.5, 0, lab)
    pos = lab != 0
    posf = pos.astype(jnp.float32)
    npos8 = jnp.sum(posf, axis=1, keepdims=True)
    n_pos = jnp.sum(npos8)

    gcx = (gx0 + gx1) / 2.0
    gcy = (gy0 + gy1) / 2.0
    gw = gx1 - gx0
    gh = gy1 - gy0
    t0 = (gcx - pcx) / (pw / 10.0)
    t1 = (gcy - pcy) / (ph / 10.0)
    t2 = jnp.log(gw / pw) * 5.0
    t3 = jnp.log(gh / ph) * 5.0
    loc_sum = (jnp.sum(jnp.abs(locs4[0] - t0) * posf) +
               jnp.sum(jnp.abs(locs4[1] - t1) * posf) +
               jnp.sum(jnp.abs(locs4[2] - t2) * posf) +
               jnp.sum(jnp.abs(locs4[3] - t3) * posf))

    m = s[0]
    picked = jnp.where(lab == 0, s[0], 0.0)
    for c in range(1, n_cls):
        sc = s[c]
        m = jnp.maximum(m, sc)
        picked = jnp.where(lab == c, sc, picked)
    se = jnp.exp(s[0] - m)
    for c in range(1, n_cls):
        se = se + jnp.exp(s[c] - m)
    lse = jnp.log(se) + m
    conf_all = lse - picked
    conf_pos = jnp.sum(conf_all * posf)
    conf_neg = jnp.where(pos, 0.0, conf_all)

    kf8 = 3.0 * npos8
    v = lax.bitcast_convert_type(conf_neg, jnp.int32)
    t8 = jnp.zeros((n_img, 1), jnp.int32)
    for b in range(30, -1, -1):
        cand = t8 | jnp.int32(1 << b)
        cnt = jnp.sum(jnp.where(v >= cand, 1.0, 0.0), axis=1,
                      keepdims=True)
        t8 = jnp.where(cnt >= kf8, cand, t8)
    tf8 = lax.bitcast_convert_type(t8, jnp.float32)
    gt = v > t8
    cnt_gt = jnp.sum(jnp.where(gt, 1.0, 0.0), axis=1, keepdims=True)
    sum_gt = jnp.sum(jnp.where(gt, conf_neg, 0.0), axis=1, keepdims=True)
    hard8 = jnp.where(kf8 > 0.0, sum_gt + (kf8 - cnt_gt) * tf8, 0.0)
    hard = jnp.sum(hard8)

    @pl.when(g == 0)
    def _init():
        acc_ref[0] = loc_sum
        acc_ref[1] = conf_pos
        acc_ref[2] = hard
        acc_ref[3] = n_pos

    @pl.when(g > 0)
    def _acc():
        acc_ref[0] += loc_sum
        acc_ref[1] += conf_pos
        acc_ref[2] += hard
        acc_ref[3] += n_pos

    n_tot = acc_ref[3]
    loss = (acc_ref[2] + acc_ref[1]) / n_tot + acc_ref[0] / (n_tot * 4.0)
    out_ref[...] = jnp.full((8, 128), loss, jnp.float32)


@jax.jit
def kernel(predicted_locs, predicted_scores, boxes, labels, priors_cxcy):
    B, P, C = predicted_scores.shape
    nobj = boxes.shape[1]
    ipg = 8
    while B % ipg:
        ipg //= 2
    locs_t = jnp.transpose(predicted_locs.astype(jnp.float32), (2, 0, 1))
    scores_t = jnp.transpose(predicted_scores.astype(jnp.float32), (2, 0, 1))
    priors_t = jnp.transpose(priors_cxcy.astype(jnp.float32), (1, 0))
    labels_i = labels.astype(jnp.int32)
    boxes_f = boxes.astype(jnp.float32)

    out = pl.pallas_call(
        _mbl_body,
        grid=(B // ipg,),
        in_specs=[
            pl.BlockSpec((ipg, nobj, 4), lambda i: (i, 0, 0)),
            pl.BlockSpec((ipg, nobj), lambda i: (i, 0)),
            pl.BlockSpec((4, P), lambda i: (0, 0)),
            pl.BlockSpec((4, ipg, P), lambda i: (0, i, 0)),
            pl.BlockSpec((C, ipg, P), lambda i: (0, i, 0)),
        ],
        out_specs=pl.BlockSpec((8, 128), lambda i: (0, 0)),
        out_shape=jax.ShapeDtypeStruct((8, 128), jnp.float32),
        scratch_shapes=[pltpu.SMEM((4,), jnp.float32)],
    )(boxes_f, labels_i, priors_t, locs_t, scores_t)
    return out[0, 0]


# speedup vs baseline: 1.1775x; 1.0812x over previous
import jax
import jax.numpy as jnp
from jax import lax
from jax.experimental import pallas as pl
from jax.experimental.pallas import tpu as pltpu


def _mbl_body(boxes_ref, labels_ref, priors_ref, locs_ref, scores_ref,
              out_ref, acc_ref):
    g = pl.program_id(0)

    boxes_g = boxes_ref[...]
    labels_g = labels_ref[...]
    pr = priors_ref[...]
    locs4 = locs_ref[...]
    s = scores_ref[...]
    n_img, nobj, _ = boxes_g.shape
    n_cls = s.shape[0]
    n_pri = s.shape[2]

    pcx = pr[0:1]
    pcy = pr[1:2]
    pw = pr[2:3]
    ph = pr[3:4]
    px0 = pcx - pw / 2.0
    py0 = pcy - ph / 2.0
    px1 = pcx + pw / 2.0
    py1 = pcy + ph / 2.0
    area_p = (px1 - px0) * (py1 - py0)

    obj_iota = lax.broadcasted_iota(jnp.int32, (nobj, n_pri), 0)
    p_iota = lax.broadcasted_iota(jnp.int32, (nobj, n_pri), 1)

    ofe_rows = []
    ovl_rows = []
    for j in range(n_img):
        bj = boxes_g[j]
        bx0 = bj[:, 0:1]
        by0 = bj[:, 1:2]
        bx1 = bj[:, 2:3]
        by1 = bj[:, 3:4]
        iw = jnp.clip(jnp.minimum(bx1, px1) - jnp.maximum(bx0, px0), 0.0)
        ih = jnp.clip(jnp.minimum(by1, py1) - jnp.maximum(by0, py0), 0.0)
        inter = iw * ih
        union = (bx1 - bx0) * (by1 - by0) + area_p - inter
        iou = inter / union

        ovl = jnp.max(iou, axis=0, keepdims=True)
        ofe = jnp.min(jnp.where(iou == ovl, obj_iota, nobj), axis=0,
                      keepdims=True)
        row_max = jnp.max(iou, axis=1, keepdims=True)
        pfo = jnp.min(jnp.where(iou == row_max, p_iota, n_pri), axis=1,
                      keepdims=True)
        hit = p_iota == pfo
        best_j = jnp.max(jnp.where(hit, obj_iota, -1), axis=0,
                         keepdims=True)
        forced = best_j >= 0
        ofe_rows.append(jnp.where(forced, best_j, ofe))
        ovl_rows.append(jnp.where(forced, 1.0, ovl))

    ofe8 = jnp.concatenate(ofe_rows, axis=0)
    ovl8 = jnp.concatenate(ovl_rows, axis=0)

    lab = jnp.zeros((n_img, n_pri), jnp.int32)
    gx0 = jnp.zeros((n_img, n_pri), jnp.float32)
    gy0 = jnp.zeros((n_img, n_pri), jnp.float32)
    gx1 = jnp.zeros((n_img, n_pri), jnp.float32)
    gy1 = jnp.zeros((n_img, n_pri), jnp.float32)
    for j in range(nobj):
        sel = ofe8 == j
        lab = jnp.where(sel, labels_g[:, j:j + 1], lab)
        gx0 = jnp.where(sel, jnp.reshape(boxes_g[:, j:j + 1, 0:1],
                                         (n_img, 1)), gx0)
        gy0 = jnp.where(sel, jnp.reshape(boxes_g[:, j:j + 1, 1:2],
                                         (n_img, 1)), gy0)
        gx1 = jnp.where(sel, jnp.reshape(boxes_g[:, j:j + 1, 2:3],
                                         (n_img, 1)), gx1)
        gy1 = jnp.where(sel, jnp.reshape(boxes_g[:, j:j + 1, 3:4],
                                         (n_img, 1)), gy1)

    lab = jnp.where(ovl8 < 0.5, 0, lab)
    pos = lab != 0
    posf = pos.astype(jnp.float32)
    npos8 = jnp.sum(posf, axis=1, keepdims=True)
    n_pos = jnp.sum(npos8)

    gcx = (gx0 + gx1) / 2.0
    gcy = (gy0 + gy1) / 2.0
    gw = gx1 - gx0
    gh = gy1 - gy0
    t0 = (gcx - pcx) / (pw / 10.0)
    t1 = (gcy - pcy) / (ph / 10.0)
    t2 = jnp.log(gw / pw) * 5.0
    t3 = jnp.log(gh / ph) * 5.0
    loc_sum = (jnp.sum(jnp.abs(locs4[0] - t0) * posf) +
               jnp.sum(jnp.abs(locs4[1] - t1) * posf) +
               jnp.sum(jnp.abs(locs4[2] - t2) * posf) +
               jnp.sum(jnp.abs(locs4[3] - t3) * posf))

    m = s[0]
    picked = jnp.where(lab == 0, s[0], 0.0)
    for c in range(1, n_cls):
        sc = s[c]
        m = jnp.maximum(m, sc)
        picked = jnp.where(lab == c, sc, picked)
    se = jnp.exp(s[0] - m)
    for c in range(1, n_cls):
        se = se + jnp.exp(s[c] - m)
    lse = jnp.log(se) + m
    conf_all = lse - picked
    conf_pos = jnp.sum(conf_all * posf)
    conf_neg = jnp.where(pos, 0.0, conf_all)

    kf8 = 3.0 * npos8
    v = lax.bitcast_convert_type(conf_neg, jnp.int32)
    t8 = jnp.zeros((n_img, 1), jnp.int32)
    for b in range(30, -1, -1):
        cand = t8 | jnp.int32(1 << b)
        cnt = jnp.sum(jnp.where(v >= cand, 1.0, 0.0), axis=1,
                      keepdims=True)
        t8 = jnp.where(cnt >= kf8, cand, t8)
    tf8 = lax.bitcast_convert_type(t8, jnp.float32)
    gt = v > t8
    cnt_gt = jnp.sum(jnp.where(gt, 1.0, 0.0), axis=1, keepdims=True)
    sum_gt = jnp.sum(jnp.where(gt, conf_neg, 0.0), axis=1, keepdims=True)
    hard8 = jnp.where(kf8 > 0.0, sum_gt + (kf8 - cnt_gt) * tf8, 0.0)
    hard = jnp.sum(hard8)

    @pl.when(g == 0)
    def _init():
        acc_ref[0] = loc_sum
        acc_ref[1] = conf_pos
        acc_ref[2] = hard
        acc_ref[3] = n_pos

    @pl.when(g > 0)
    def _acc():
        acc_ref[0] += loc_sum
        acc_ref[1] += conf_pos
        acc_ref[2] += hard
        acc_ref[3] += n_pos

    n_tot = acc_ref[3]
    loss = (acc_ref[2] + acc_ref[1]) / n_tot + acc_ref[0] / (n_tot * 4.0)
    out_ref[...] = jnp.full((8, 128), loss, jnp.float32)


@jax.jit
def kernel(predicted_locs, predicted_scores, boxes, labels, priors_cxcy):
    B, P, C = predicted_scores.shape
    nobj = boxes.shape[1]
    ipg = 16
    while B % ipg:
        ipg //= 2
    locs_t = jnp.transpose(predicted_locs.astype(jnp.float32), (2, 0, 1))
    scores_t = jnp.transpose(predicted_scores.astype(jnp.float32), (2, 0, 1))
    priors_t = jnp.transpose(priors_cxcy.astype(jnp.float32), (1, 0))
    labels_i = labels.astype(jnp.int32)
    boxes_f = boxes.astype(jnp.float32)

    out = pl.pallas_call(
        _mbl_body,
        grid=(B // ipg,),
        in_specs=[
            pl.BlockSpec((ipg, nobj, 4), lambda i: (i, 0, 0)),
            pl.BlockSpec((ipg, nobj), lambda i: (i, 0)),
            pl.BlockSpec((4, P), lambda i: (0, 0)),
            pl.BlockSpec((4, ipg, P), lambda i: (0, i, 0)),
            pl.BlockSpec((C, ipg, P), lambda i: (0, i, 0)),
        ],
        out_specs=pl.BlockSpec((8, 128), lambda i: (0, 0)),
        out_shape=jax.ShapeDtypeStruct((8, 128), jnp.float32),
        scratch_shapes=[pltpu.SMEM((4,), jnp.float32)],
    )(boxes_f, labels_i, priors_t, locs_t, scores_t)
    return out[0, 0]
